# baseline (speedup 1.0000x reference)
import functools

import jax
import jax.numpy as jnp
import numpy as np
from jax.experimental import pallas as pl
from jax.experimental.pallas import tpu as pltpu

_NS = 8
_NEG_INF = np.float32(-np.inf)
_TINY = np.float32(np.finfo(np.float32).tiny)
_LOG_CLAMP = np.float32(np.log(1e-12))


def _threefry_bits(x1):
    ks0 = np.uint32(0)
    ks1 = np.uint32(42)
    ks2 = np.uint32(ks0 ^ ks1 ^ np.uint32(0x1BD11BDA))

    def rotl(x, d):
        return (x << np.uint32(d)) | (x >> np.uint32(32 - d))

    def rounds(x0, x1, rots):
        for r in rots:
            x0 = x0 + x1
            x1 = rotl(x1, r)
            x1 = x0 ^ x1
        return x0, x1

    x0 = x1
    x1 = rotl(x1, 13) ^ x1
    x0, x1 = rounds(x0, x1, (15, 26, 6))
    x0 = x0 + ks1
    x1 = x1 + (ks2 + np.uint32(1))
    x0, x1 = rounds(x0, x1, (17, 29, 16, 24))
    x0 = x0 + ks2
    x1 = x1 + (ks0 + np.uint32(2))
    x0, x1 = rounds(x0, x1, (13, 15, 26, 6))
    x0 = x0 + ks0
    x1 = x1 + (ks1 + np.uint32(3))
    x0, x1 = rounds(x0, x1, (17, 29, 16, 24))
    x0 = x0 + ks1
    x1 = x1 + (ks2 + np.uint32(4))
    x0, x1 = rounds(x0, x1, (13, 15, 26, 6))
    x0 = x0 + ks2
    x1 = x1 + (ks0 + np.uint32(5))
    return x0 ^ x1


def _gumbel_from_bits(bits):
    fb = (bits >> np.uint32(9)) | np.uint32(0x3F800000)
    f = jax.lax.bitcast_convert_type(fb, jnp.float32) - np.float32(1.0)
    u = jnp.maximum(_TINY, f + _TINY)
    return -jnp.log(-jnp.log(u))


def _sample_kernel(logits_ref, chosen_ref, scores_ref, logp_ref,
                   bz_ref, bi_ref, bl_ref, m_ref, s_ref,
                   *, b_rows, v_cols, c_chunk, n_chunks):
    j = pl.program_id(0)

    @pl.when(j == 0)
    def _init():
        bz_ref[...] = jnp.full((b_rows, _NS), _NEG_INF, jnp.float32)
        bi_ref[...] = jnp.zeros((b_rows, _NS), jnp.int32)
        bl_ref[...] = jnp.zeros((b_rows, _NS), jnp.float32)
        m_ref[...] = jnp.full((b_rows, 1), _NEG_INF, jnp.float32)
        s_ref[...] = jnp.zeros((b_rows, 1), jnp.float32)

    lb = logits_ref[...]
    col = jax.lax.broadcasted_iota(jnp.int32, (b_rows, c_chunk), 1) + j * c_chunk
    row_base = jax.lax.broadcasted_iota(jnp.int32, (b_rows, c_chunk), 0) * v_cols
    n_base = col + row_base

    def scan_block(masked):
        if masked:
            valid = col < v_cols
            lbm = jnp.where(valid, lb, _NEG_INF)
        else:
            lbm = lb

        m_old = m_ref[...]
        m_new = jnp.maximum(m_old, jnp.max(lbm, axis=1, keepdims=True))
        e = jnp.exp(lbm - m_new)
        s_ref[...] = s_ref[...] * jnp.exp(m_old - m_new) + jnp.sum(
            e, axis=1, keepdims=True)
        m_ref[...] = m_new

        for s in range(_NS):
            x1 = (n_base + np.int32(s * b_rows * v_cols + 42)).astype(jnp.uint32)
            g = _gumbel_from_bits(_threefry_bits(x1))
            z = g + lbm
            zmax = jnp.max(z, axis=1, keepdims=True)
            eq = z == zmax
            idx = jnp.min(jnp.where(eq, col, np.int32(0x7FFFFFFF)), axis=1,
                          keepdims=True)
            lat = jnp.max(jnp.where(col == idx, lb, _NEG_INF), axis=1,
                          keepdims=True)
            better = zmax > bz_ref[:, s:s + 1]
            bz_ref[:, s:s + 1] = jnp.where(better, zmax, bz_ref[:, s:s + 1])
            bi_ref[:, s:s + 1] = jnp.where(better, idx, bi_ref[:, s:s + 1])
            bl_ref[:, s:s + 1] = jnp.where(better, lat, bl_ref[:, s:s + 1])

    if v_cols % c_chunk == 0:
        scan_block(False)
    else:
        @pl.when(j < n_chunks - 1)
        def _full():
            scan_block(False)

        @pl.when(j == n_chunks - 1)
        def _tail():
            scan_block(True)

    @pl.when(j == n_chunks - 1)
    def _finish():
        chosen_ref[...] = bi_ref[...]
        scores_ref[...] = bl_ref[...]
        log_z = m_ref[...] + jnp.log(s_ref[...])
        logp_ref[...] = jnp.maximum(bl_ref[...] - log_z, _LOG_CLAMP)


@jax.jit
def kernel(logits):
    b_rows, v_cols = logits.shape
    c_chunk = 2048
    n_chunks = -(-v_cols // c_chunk)

    body = functools.partial(_sample_kernel, b_rows=b_rows, v_cols=v_cols,
                             c_chunk=c_chunk, n_chunks=n_chunks)
    chosen, scores, logp = pl.pallas_call(
        body,
        grid=(n_chunks,),
        in_specs=[pl.BlockSpec((b_rows, c_chunk), lambda j: (0, j))],
        out_specs=[
            pl.BlockSpec((b_rows, _NS), lambda j: (0, 0)),
            pl.BlockSpec((b_rows, _NS), lambda j: (0, 0)),
            pl.BlockSpec((b_rows, _NS), lambda j: (0, 0)),
        ],
        out_shape=[
            jax.ShapeDtypeStruct((b_rows, _NS), jnp.int32),
            jax.ShapeDtypeStruct((b_rows, _NS), jnp.float32),
            jax.ShapeDtypeStruct((b_rows, _NS), jnp.float32),
        ],
        scratch_shapes=[
            pltpu.VMEM((b_rows, _NS), jnp.float32),
            pltpu.VMEM((b_rows, _NS), jnp.int32),
            pltpu.VMEM((b_rows, _NS), jnp.float32),
            pltpu.VMEM((b_rows, 1), jnp.float32),
            pltpu.VMEM((b_rows, 1), jnp.float32),
        ],
        compiler_params=pltpu.CompilerParams(
            dimension_semantics=("arbitrary",),
        ),
    )(logits)
    return (chosen, scores, logp)


# speedup vs baseline: 1.9818x; 1.9818x over previous
import functools

import jax
import jax.numpy as jnp
import numpy as np
from jax.experimental import pallas as pl
from jax.experimental.pallas import tpu as pltpu

_NS = 8
_NEG_INF = np.float32(-np.inf)
_TINY = np.float32(np.finfo(np.float32).tiny)
_LOG_CLAMP = np.float32(np.log(1e-12))


def _threefry_bits(x1):
    ks0 = np.uint32(0)
    ks1 = np.uint32(42)
    ks2 = np.uint32(ks0 ^ ks1 ^ np.uint32(0x1BD11BDA))

    def rotl(x, d):
        return (x << np.uint32(d)) | (x >> np.uint32(32 - d))

    def rounds(x0, x1, rots):
        for r in rots:
            x0 = x0 + x1
            x1 = rotl(x1, r)
            x1 = x0 ^ x1
        return x0, x1

    x0 = x1
    x1 = rotl(x1, 13) ^ x1
    x0, x1 = rounds(x0, x1, (15, 26, 6))
    x0 = x0 + ks1
    x1 = x1 + (ks2 + np.uint32(1))
    x0, x1 = rounds(x0, x1, (17, 29, 16, 24))
    x0 = x0 + ks2
    x1 = x1 + (ks0 + np.uint32(2))
    x0, x1 = rounds(x0, x1, (13, 15, 26, 6))
    x0 = x0 + ks0
    x1 = x1 + (ks1 + np.uint32(3))
    x0, x1 = rounds(x0, x1, (17, 29, 16, 24))
    x0 = x0 + ks1
    x1 = x1 + (ks2 + np.uint32(4))
    x0, x1 = rounds(x0, x1, (13, 15, 26, 6))
    x0 = x0 + ks2
    x1 = x1 + (ks0 + np.uint32(5))
    return x0 ^ x1


def _gumbel_from_bits(bits):
    fb = (bits >> np.uint32(9)) | np.uint32(0x3F800000)
    f = jax.lax.bitcast_convert_type(fb, jnp.float32) - np.float32(1.0)
    u = jnp.maximum(_TINY, f + _TINY)
    return -jnp.log(-jnp.log(u))


def _sample_kernel(logits_ref, chosen_ref, scores_ref, logp_ref,
                   bz_ref, bi_ref, bl_ref, m_ref, s_ref,
                   *, b_rows, v_cols, c_chunk, n_chunks):
    j = pl.program_id(0)

    @pl.when(j == 0)
    def _init():
        bz_ref[...] = jnp.full((b_rows, _NS), _NEG_INF, jnp.float32)
        bi_ref[...] = jnp.zeros((b_rows, _NS), jnp.int32)
        bl_ref[...] = jnp.zeros((b_rows, _NS), jnp.float32)
        m_ref[...] = jnp.full((b_rows, 1), _NEG_INF, jnp.float32)
        s_ref[...] = jnp.zeros((b_rows, 1), jnp.float32)

    lb = logits_ref[...]
    col = jax.lax.broadcasted_iota(jnp.int32, (b_rows, c_chunk), 1) + j * c_chunk
    row_base = jax.lax.broadcasted_iota(jnp.int32, (b_rows, c_chunk), 0) * v_cols
    n_base = col + row_base

    def scan_block(masked):
        if masked:
            valid = col < v_cols
            lbm = jnp.where(valid, lb, _NEG_INF)
        else:
            lbm = lb

        m_old = m_ref[...]
        m_new = jnp.maximum(m_old, jnp.max(lbm, axis=1, keepdims=True))
        e = jnp.exp(lbm - m_new)
        s_ref[...] = s_ref[...] * jnp.exp(m_old - m_new) + jnp.sum(
            e, axis=1, keepdims=True)
        m_ref[...] = m_new

        for s in range(_NS):
            x1 = (n_base + np.int32(s * b_rows * v_cols + 42)).astype(jnp.uint32)
            g = _gumbel_from_bits(_threefry_bits(x1))
            z = g + lbm
            zmax = jnp.max(z, axis=1, keepdims=True)
            eq = z == zmax
            idx = jnp.min(jnp.where(eq, col, np.int32(0x7FFFFFFF)), axis=1,
                          keepdims=True)
            lat = jnp.max(jnp.where(col == idx, lb, _NEG_INF), axis=1,
                          keepdims=True)
            better = zmax > bz_ref[:, s:s + 1]
            bz_ref[:, s:s + 1] = jnp.where(better, zmax, bz_ref[:, s:s + 1])
            bi_ref[:, s:s + 1] = jnp.where(better, idx, bi_ref[:, s:s + 1])
            bl_ref[:, s:s + 1] = jnp.where(better, lat, bl_ref[:, s:s + 1])

    scan_block(masked=(v_cols % c_chunk != 0))

    @pl.when(j == n_chunks - 1)
    def _finish():
        chosen_ref[...] = bi_ref[...]
        scores_ref[...] = bl_ref[...]
        log_z = m_ref[...] + jnp.log(s_ref[...])
        logp_ref[...] = jnp.maximum(bl_ref[...] - log_z, _LOG_CLAMP)


@jax.jit
def kernel(logits):
    b_rows, v_cols = logits.shape
    c_chunk = 2048
    n_chunks = -(-v_cols // c_chunk)

    body = functools.partial(_sample_kernel, b_rows=b_rows, v_cols=v_cols,
                             c_chunk=c_chunk, n_chunks=n_chunks)
    chosen, scores, logp = pl.pallas_call(
        body,
        grid=(n_chunks,),
        in_specs=[pl.BlockSpec((b_rows, c_chunk), lambda j: (0, j))],
        out_specs=[
            pl.BlockSpec((b_rows, _NS), lambda j: (0, 0)),
            pl.BlockSpec((b_rows, _NS), lambda j: (0, 0)),
            pl.BlockSpec((b_rows, _NS), lambda j: (0, 0)),
        ],
        out_shape=[
            jax.ShapeDtypeStruct((b_rows, _NS), jnp.int32),
            jax.ShapeDtypeStruct((b_rows, _NS), jnp.float32),
            jax.ShapeDtypeStruct((b_rows, _NS), jnp.float32),
        ],
        scratch_shapes=[
            pltpu.VMEM((b_rows, _NS), jnp.float32),
            pltpu.VMEM((b_rows, _NS), jnp.int32),
            pltpu.VMEM((b_rows, _NS), jnp.float32),
            pltpu.VMEM((b_rows, 1), jnp.float32),
            pltpu.VMEM((b_rows, 1), jnp.float32),
        ],
        compiler_params=pltpu.CompilerParams(
            dimension_semantics=("arbitrary",),
        ),
    )(logits)
    return (chosen, scores, logp)


# speedup vs baseline: 2.0321x; 1.0254x over previous
import functools

import jax
import jax.numpy as jnp
import numpy as np
from jax.experimental import pallas as pl
from jax.experimental.pallas import tpu as pltpu

_NS = 8
_NEG_INF = np.float32(-np.inf)
_TINY = np.float32(np.finfo(np.float32).tiny)
_LOG_CLAMP = np.float32(np.log(1e-12))


def _threefry_bits(x1):
    ks0 = np.uint32(0)
    ks1 = np.uint32(42)
    ks2 = np.uint32(ks0 ^ ks1 ^ np.uint32(0x1BD11BDA))

    def rotl(x, d):
        return (x << np.uint32(d)) | (x >> np.uint32(32 - d))

    def rounds(x0, x1, rots):
        for r in rots:
            x0 = x0 + x1
            x1 = rotl(x1, r)
            x1 = x0 ^ x1
        return x0, x1

    x0 = x1
    x1 = rotl(x1, 13) ^ x1
    x0, x1 = rounds(x0, x1, (15, 26, 6))
    x0 = x0 + ks1
    x1 = x1 + (ks2 + np.uint32(1))
    x0, x1 = rounds(x0, x1, (17, 29, 16, 24))
    x0 = x0 + ks2
    x1 = x1 + (ks0 + np.uint32(2))
    x0, x1 = rounds(x0, x1, (13, 15, 26, 6))
    x0 = x0 + ks0
    x1 = x1 + (ks1 + np.uint32(3))
    x0, x1 = rounds(x0, x1, (17, 29, 16, 24))
    x0 = x0 + ks1
    x1 = x1 + (ks2 + np.uint32(4))
    x0, x1 = rounds(x0, x1, (13, 15, 26, 6))
    x0 = x0 + ks2
    x1 = x1 + (ks0 + np.uint32(5))
    return x0 ^ x1


def _gumbel_from_bits(bits):
    fb = (bits >> np.uint32(9)) | np.uint32(0x3F800000)
    f = jax.lax.bitcast_convert_type(fb, jnp.float32) - np.float32(1.0)
    u = jnp.maximum(_TINY, f + _TINY)
    return -jnp.log(-jnp.log(u))


def _sample_kernel(logits_ref, chosen_ref, scores_ref, logp_ref,
                   bz_ref, bi_ref, bl_ref, m_ref, s_ref,
                   *, b_rows, v_cols, c_chunk, n_chunks):
    j = pl.program_id(0)

    @pl.when(j == 0)
    def _init():
        bz_ref[...] = jnp.full((b_rows, _NS), _NEG_INF, jnp.float32)
        bi_ref[...] = jnp.zeros((b_rows, _NS), jnp.int32)
        bl_ref[...] = jnp.zeros((b_rows, _NS), jnp.float32)
        m_ref[...] = jnp.full((b_rows, 1), _NEG_INF, jnp.float32)
        s_ref[...] = jnp.zeros((b_rows, 1), jnp.float32)

    lb = logits_ref[...]
    col_l = jax.lax.broadcasted_iota(jnp.int32, (b_rows, c_chunk), 1)
    row_base = jax.lax.broadcasted_iota(jnp.int32, (b_rows, c_chunk), 0) * v_cols
    n_base = col_l + row_base

    def scan_block(masked):
        if masked:
            valid = col_l < v_cols - j * c_chunk
            lbm = jnp.where(valid, lb, _NEG_INF)
        else:
            lbm = lb

        m_old = m_ref[...]
        m_new = jnp.maximum(m_old, jnp.max(lbm, axis=1, keepdims=True))
        e = jnp.exp(lbm - m_new)
        s_ref[...] = s_ref[...] * jnp.exp(m_old - m_new) + jnp.sum(
            e, axis=1, keepdims=True)
        m_ref[...] = m_new

        for s in range(_NS):
            x1 = (n_base + (j * c_chunk + np.int32(s * b_rows * v_cols + 42))
                  ).astype(jnp.uint32)
            g = _gumbel_from_bits(_threefry_bits(x1))
            z = g + lbm
            zmax = jnp.max(z, axis=1, keepdims=True)
            eq = z == zmax
            idx_l = jnp.min(jnp.where(eq, col_l, np.int32(0x7FFFFFFF)),
                            axis=1, keepdims=True)
            lat = jnp.max(jnp.where(col_l == idx_l, lb, _NEG_INF), axis=1,
                          keepdims=True)
            better = zmax > bz_ref[:, s:s + 1]
            bz_ref[:, s:s + 1] = jnp.where(better, zmax, bz_ref[:, s:s + 1])
            bi_ref[:, s:s + 1] = jnp.where(better, idx_l + j * c_chunk,
                                           bi_ref[:, s:s + 1])
            bl_ref[:, s:s + 1] = jnp.where(better, lat, bl_ref[:, s:s + 1])

    scan_block(masked=(v_cols % c_chunk != 0))

    @pl.when(j == n_chunks - 1)
    def _finish():
        chosen_ref[...] = bi_ref[...]
        scores_ref[...] = bl_ref[...]
        log_z = m_ref[...] + jnp.log(s_ref[...])
        logp_ref[...] = jnp.maximum(bl_ref[...] - log_z, _LOG_CLAMP)


@jax.jit
def kernel(logits):
    b_rows, v_cols = logits.shape
    c_chunk = 1024
    n_chunks = -(-v_cols // c_chunk)

    body = functools.partial(_sample_kernel, b_rows=b_rows, v_cols=v_cols,
                             c_chunk=c_chunk, n_chunks=n_chunks)
    chosen, scores, logp = pl.pallas_call(
        body,
        grid=(n_chunks,),
        in_specs=[pl.BlockSpec((b_rows, c_chunk), lambda j: (0, j))],
        out_specs=[
            pl.BlockSpec((b_rows, _NS), lambda j: (0, 0)),
            pl.BlockSpec((b_rows, _NS), lambda j: (0, 0)),
            pl.BlockSpec((b_rows, _NS), lambda j: (0, 0)),
        ],
        out_shape=[
            jax.ShapeDtypeStruct((b_rows, _NS), jnp.int32),
            jax.ShapeDtypeStruct((b_rows, _NS), jnp.float32),
            jax.ShapeDtypeStruct((b_rows, _NS), jnp.float32),
        ],
        scratch_shapes=[
            pltpu.VMEM((b_rows, _NS), jnp.float32),
            pltpu.VMEM((b_rows, _NS), jnp.int32),
            pltpu.VMEM((b_rows, _NS), jnp.float32),
            pltpu.VMEM((b_rows, 1), jnp.float32),
            pltpu.VMEM((b_rows, 1), jnp.float32),
        ],
        compiler_params=pltpu.CompilerParams(
            dimension_semantics=("arbitrary",),
        ),
    )(logits)
    return (chosen, scores, logp)


# speedup vs baseline: 2.0867x; 1.0269x over previous
import functools

import jax
import jax.numpy as jnp
import numpy as np
from jax.experimental import pallas as pl
from jax.experimental.pallas import tpu as pltpu

_NS = 8
_NEG_INF = np.float32(-np.inf)
_TINY = np.float32(np.finfo(np.float32).tiny)
_LOG_CLAMP = np.float32(np.log(1e-12))


def _threefry_bits(x1):
    ks0 = np.uint32(0)
    ks1 = np.uint32(42)
    ks2 = np.uint32(ks0 ^ ks1 ^ np.uint32(0x1BD11BDA))

    def rotl(x, d):
        return (x << np.uint32(d)) | (x >> np.uint32(32 - d))

    def rounds(x0, x1, rots):
        for r in rots:
            x0 = x0 + x1
            x1 = rotl(x1, r)
            x1 = x0 ^ x1
        return x0, x1

    x0 = x1
    x1 = rotl(x1, 13) ^ x1
    x0, x1 = rounds(x0, x1, (15, 26, 6))
    x0 = x0 + ks1
    x1 = x1 + (ks2 + np.uint32(1))
    x0, x1 = rounds(x0, x1, (17, 29, 16, 24))
    x0 = x0 + ks2
    x1 = x1 + (ks0 + np.uint32(2))
    x0, x1 = rounds(x0, x1, (13, 15, 26, 6))
    x1 = x1 + (ks1 + np.uint32(3))
    x0, x1 = rounds(x0, x1, (17, 29, 16, 24))
    x0 = x0 + ks1
    x1 = x1 + (ks2 + np.uint32(4))
    x0, x1 = rounds(x0, x1, (13, 15, 26, 6))
    x0 = x0 + ks2
    x1 = x1 + (ks0 + np.uint32(5))
    return x0 ^ x1


def _gumbel_from_bits(bits):
    fb = (bits >> np.uint32(9)) | np.uint32(0x3F800000)
    f = jax.lax.bitcast_convert_type(fb, jnp.float32) - np.float32(1.0)
    u = jnp.maximum(_TINY, f + _TINY)
    return -jnp.log(-jnp.log(u))


def _sample_kernel(logits_ref, chosen_ref, scores_ref, logp_ref,
                   bz_ref, bi_ref, m_ref, s_ref,
                   *, b_rows, v_cols, c_chunk, n_chunks):
    j = pl.program_id(0)

    @pl.when(j == 0)
    def _init():
        bz_ref[...] = jnp.full((b_rows, _NS), _NEG_INF, jnp.float32)
        bi_ref[...] = jnp.zeros((b_rows, _NS), jnp.int32)
        m_ref[...] = jnp.full((b_rows, 1), _NEG_INF, jnp.float32)
        s_ref[...] = jnp.zeros((b_rows, 1), jnp.float32)

    lb = logits_ref[...]
    col_l = jax.lax.broadcasted_iota(jnp.int32, (b_rows, c_chunk), 1)
    row_base = jax.lax.broadcasted_iota(jnp.int32, (b_rows, c_chunk), 0) * v_cols
    n_base = col_l + row_base

    def scan_block(masked):
        if masked:
            valid = col_l < v_cols - j * c_chunk
            lbm = jnp.where(valid, lb, _NEG_INF)
        else:
            lbm = lb

        m_old = m_ref[...]
        m_new = jnp.maximum(m_old, jnp.max(lbm, axis=1, keepdims=True))
        e = jnp.exp(lbm - m_new)
        s_ref[...] = s_ref[...] * jnp.exp(m_old - m_new) + jnp.sum(
            e, axis=1, keepdims=True)
        m_ref[...] = m_new

        for s in range(_NS):
            x1 = (n_base + (j * c_chunk + np.int32(s * b_rows * v_cols + 42))
                  ).astype(jnp.uint32)
            g = _gumbel_from_bits(_threefry_bits(x1))
            z = g + lbm
            zmax = jnp.max(z, axis=1, keepdims=True)
            eq = z == zmax
            idx_l = jnp.min(jnp.where(eq, col_l, np.int32(0x7FFFFFFF)),
                            axis=1, keepdims=True)
            better = zmax > bz_ref[:, s:s + 1]
            bz_ref[:, s:s + 1] = jnp.where(better, zmax, bz_ref[:, s:s + 1])
            bi_ref[:, s:s + 1] = jnp.where(better, idx_l + j * c_chunk,
                                           bi_ref[:, s:s + 1])

    scan_block(masked=(v_cols % c_chunk != 0))

    @pl.when(j == n_chunks - 1)
    def _finish():
        bi = bi_ref[...]
        chosen_ref[...] = bi
        r_iota = jax.lax.broadcasted_iota(jnp.int32, (b_rows, _NS), 0)
        s_iota = jax.lax.broadcasted_iota(jnp.int32, (b_rows, _NS), 1)
        n = (s_iota * np.int32(b_rows * v_cols) + r_iota * np.int32(v_cols)
             + bi + np.int32(42)).astype(jnp.uint32)
        g_star = _gumbel_from_bits(_threefry_bits(n))
        scores = bz_ref[...] - g_star
        scores_ref[...] = scores
        log_z = m_ref[...] + jnp.log(s_ref[...])
        logp_ref[...] = jnp.maximum(scores - log_z, _LOG_CLAMP)


@jax.jit
def kernel(logits):
    b_rows, v_cols = logits.shape
    c_chunk = 1024
    n_chunks = -(-v_cols // c_chunk)

    body = functools.partial(_sample_kernel, b_rows=b_rows, v_cols=v_cols,
                             c_chunk=c_chunk, n_chunks=n_chunks)
    chosen, scores, logp = pl.pallas_call(
        body,
        grid=(n_chunks,),
        in_specs=[pl.BlockSpec((b_rows, c_chunk), lambda j: (0, j))],
        out_specs=[
            pl.BlockSpec((b_rows, _NS), lambda j: (0, 0)),
            pl.BlockSpec((b_rows, _NS), lambda j: (0, 0)),
            pl.BlockSpec((b_rows, _NS), lambda j: (0, 0)),
        ],
        out_shape=[
            jax.ShapeDtypeStruct((b_rows, _NS), jnp.int32),
            jax.ShapeDtypeStruct((b_rows, _NS), jnp.float32),
            jax.ShapeDtypeStruct((b_rows, _NS), jnp.float32),
        ],
        scratch_shapes=[
            pltpu.VMEM((b_rows, _NS), jnp.float32),
            pltpu.VMEM((b_rows, _NS), jnp.int32),
            pltpu.VMEM((b_rows, 1), jnp.float32),
            pltpu.VMEM((b_rows, 1), jnp.float32),
        ],
        compiler_params=pltpu.CompilerParams(
            dimension_semantics=("arbitrary",),
        ),
    )(logits)
    return (chosen, scores, logp)
